# Initial kernel scaffold; baseline (speedup 1.0000x reference)
import functools

import jax
import jax.numpy as jnp
from jax import lax
from jax.experimental import pallas as pl
from jax.experimental.pallas import tpu as pltpu
from jax.experimental.pallas import tpu_sc as plsc

N = 100000
D = 128
S = 16

NC = 2
NS = 16
NW = NC * NS

CHUNK = 8
TOTAL_CHUNKS = N // CHUNK
Q, R = divmod(TOTAL_CHUNKS, NW)
MAXC = Q + 1


def _gather_sum_body(feat_hbm, idx_hbm, out_hbm, idx_v, rows_v, self_v, out_v,
                     sem_self, sem_rows):
    wid = lax.axis_index("s") * NC + lax.axis_index("c")
    chunk_base = wid * Q + lax.min(wid, R)
    nchunks = Q + jnp.where(wid < R, 1, 0)

    pbase = lax.min(chunk_base, TOTAL_CHUNKS - MAXC)
    off = chunk_base - pbase
    pltpu.sync_copy(idx_hbm.at[pl.ds(pbase, MAXC)], idx_v)

    def body(j, _):
        node_base = (chunk_base + j) * CHUNK
        cp_self = pltpu.async_copy(
            feat_hbm.at[pl.ds(node_base, CHUNK)], self_v, sem_self)
        cp_rows = pltpu.async_copy(
            feat_hbm.at[idx_v.at[off + j]], rows_v, sem_rows)
        cp_self.wait()
        cp_rows.wait()
        for c in range(CHUNK):
            for dk in range(D // 16):
                sl = pl.ds(dk * 16, 16)
                acc = self_v[c, sl]
                for s in range(S):
                    acc = acc + rows_v[c * S + s, sl]
                out_v[c, sl] = acc
        pltpu.sync_copy(out_v, out_hbm.at[pl.ds(node_base, CHUNK)])
        return 0

    lax.fori_loop(0, nchunks, body, 0)


_gather_sum = functools.partial(
    pl.kernel,
    out_type=jax.ShapeDtypeStruct((N, D), jnp.float32),
    mesh=plsc.VectorSubcoreMesh(core_axis_name="c", subcore_axis_name="s"),
    scratch_types=[
        pltpu.VMEM((MAXC, CHUNK * S), jnp.int32),
        pltpu.VMEM((CHUNK * S, D), jnp.float32),
        pltpu.VMEM((CHUNK, D), jnp.float32),
        pltpu.VMEM((CHUNK, D), jnp.float32),
        pltpu.SemaphoreType.DMA,
        pltpu.SemaphoreType.DMA,
    ],
)(_gather_sum_body)


BN = 1000


def _norm_linear_body(h_ref, wt_ref, b_ref, out_ref):
    h = h_ref[...]
    ss = jnp.sum(h * h, axis=1, keepdims=True)
    denom = jnp.maximum(jnp.sqrt(ss), 1e-12)
    hn = h / denom
    out_ref[...] = (
        jnp.dot(hn, wt_ref[...], preferred_element_type=jnp.float32)
        + b_ref[...]
    )


def _norm_linear(h, wt, b2d):
    return pl.pallas_call(
        _norm_linear_body,
        grid=(N // BN,),
        in_specs=[
            pl.BlockSpec((BN, D), lambda i: (i, 0)),
            pl.BlockSpec((D, D), lambda i: (0, 0)),
            pl.BlockSpec((1, D), lambda i: (0, 0)),
        ],
        out_specs=pl.BlockSpec((BN, D), lambda i: (i, 0)),
        out_shape=jax.ShapeDtypeStruct((N, D), jnp.float32),
    )(h, wt, b2d)


def kernel(features, non_neighbor_idx, W, b):
    idx = non_neighbor_idx.astype(jnp.int32).reshape(TOTAL_CHUNKS, CHUNK * S)
    h = _gather_sum(features, idx)
    return _norm_linear(h, W.T, b.reshape(1, D))


# speedup vs baseline: 2.4698x; 2.4698x over previous
import functools

import jax
import jax.numpy as jnp
from jax import lax
from jax.experimental import pallas as pl
from jax.experimental.pallas import tpu as pltpu
from jax.experimental.pallas import tpu_sc as plsc

N = 100000
D = 128
S = 16

NC = 2
NS = 16
NW = NC * NS

CHUNK = 8
TOTAL_CHUNKS = N // CHUNK
Q, R = divmod(TOTAL_CHUNKS, NW)
MAXC = 400
PADC = 12504


def _gather_sum_body(feat_hbm, idx_hbm, out_hbm, idx_v, rows_v, self_v, out_v,
                     sem_self, sem_rows):
    wid = lax.axis_index("s") * NC + lax.axis_index("c")
    chunk_base = wid * Q + lax.min(wid, R)
    nchunks = Q + jnp.where(wid < R, 1, 0)

    pbase = (chunk_base // 8) * 8
    off = chunk_base - pbase
    pltpu.sync_copy(idx_hbm.at[pl.ds(pbase, MAXC)], idx_v)

    def body(j, _):
        node_base = (chunk_base + j) * CHUNK
        cp_self = pltpu.async_copy(
            feat_hbm.at[pl.ds(node_base, CHUNK)], self_v, sem_self)
        cp_rows = pltpu.async_copy(
            feat_hbm.at[idx_v.at[off + j]], rows_v, sem_rows)
        cp_self.wait()
        cp_rows.wait()
        for c in range(CHUNK):
            for dk in range(D // 16):
                sl = pl.ds(dk * 16, 16)
                acc = self_v[c, sl]
                for s in range(S):
                    acc = acc + rows_v[c * S + s, sl]
                out_v[c, sl] = acc
        pltpu.sync_copy(out_v, out_hbm.at[pl.ds(node_base, CHUNK)])
        return 0

    lax.fori_loop(0, nchunks, body, 0)


_gather_sum = functools.partial(
    pl.kernel,
    out_type=jax.ShapeDtypeStruct((N, D), jnp.float32),
    mesh=plsc.VectorSubcoreMesh(core_axis_name="c", subcore_axis_name="s"),
    scratch_types=[
        pltpu.VMEM((MAXC, CHUNK * S), jnp.int32),
        pltpu.VMEM((CHUNK * S, D), jnp.float32),
        pltpu.VMEM((CHUNK, D), jnp.float32),
        pltpu.VMEM((CHUNK, D), jnp.float32),
        pltpu.SemaphoreType.DMA,
        pltpu.SemaphoreType.DMA,
    ],
)(_gather_sum_body)


BN = 1000


def _norm_linear_body(h_ref, wt_ref, b_ref, out_ref):
    h = h_ref[...]
    ss = jnp.sum(h * h, axis=1, keepdims=True)
    denom = jnp.maximum(jnp.sqrt(ss), 1e-12)
    hn = h / denom
    out_ref[...] = (
        jnp.dot(hn, wt_ref[...], preferred_element_type=jnp.float32)
        + b_ref[...]
    )


def _norm_linear(h, wt, b2d):
    return pl.pallas_call(
        _norm_linear_body,
        grid=(N // BN,),
        in_specs=[
            pl.BlockSpec((BN, D), lambda i: (i, 0)),
            pl.BlockSpec((D, D), lambda i: (0, 0)),
            pl.BlockSpec((1, D), lambda i: (0, 0)),
        ],
        out_specs=pl.BlockSpec((BN, D), lambda i: (i, 0)),
        out_shape=jax.ShapeDtypeStruct((N, D), jnp.float32),
    )(h, wt, b2d)


def kernel(features, non_neighbor_idx, W, b):
    idx = non_neighbor_idx.astype(jnp.int32).reshape(TOTAL_CHUNKS, CHUNK * S)
    idx = jnp.pad(idx, ((0, PADC - TOTAL_CHUNKS), (0, 0)))
    h = _gather_sum(features, idx)
    return _norm_linear(h, W.T, b.reshape(1, D))


# speedup vs baseline: 3.5240x; 1.4268x over previous
import functools

import jax
import jax.numpy as jnp
from jax import lax
from jax.experimental import pallas as pl
from jax.experimental.pallas import tpu as pltpu
from jax.experimental.pallas import tpu_sc as plsc

N = 100000
D = 128
S = 16

NC = 2
NS = 16
NW = NC * NS

CHUNK = 8
TOTAL_CHUNKS = N // CHUNK
Q, R = divmod(TOTAL_CHUNKS, NW)
MAXC = 400
PADC = 12504


def _gather_sum_body(feat_hbm, idx_hbm, out_hbm, idx_v,
                     rows0, rows1, self0, self1, out0, out1,
                     sr0, sr1, ss0, ss1, so0, so1):
    wid = lax.axis_index("s") * NC + lax.axis_index("c")
    chunk_base = wid * Q + lax.min(wid, R)
    nchunks = Q + jnp.where(wid < R, 1, 0)

    pbase = (chunk_base // 8) * 8
    off = chunk_base - pbase
    pltpu.sync_copy(idx_hbm.at[pl.ds(pbase, MAXC)], idx_v)

    rows = (rows0, rows1)
    selfs = (self0, self1)
    outs = (out0, out1)
    sems_r = (sr0, sr1)
    sems_s = (ss0, ss1)
    sems_o = (so0, so1)

    def gathers(j, p):
        node_base = (chunk_base + j) * CHUNK
        return (
            pltpu.make_async_copy(
                feat_hbm.at[pl.ds(node_base, CHUNK)], selfs[p], sems_s[p]),
            pltpu.make_async_copy(
                feat_hbm.at[idx_v.at[off + j]], rows[p], sems_r[p]),
        )

    def out_store(j, p):
        node_base = (chunk_base + j) * CHUNK
        return pltpu.make_async_copy(
            outs[p], out_hbm.at[pl.ds(node_base, CHUNK)], sems_o[p])

    def fire(j, p):
        for cp in gathers(j, p):
            cp.start()

    def consume(j, jj, p):
        for cp in gathers(j, p):
            cp.wait()

        @pl.when(jj >= 1)
        def _():
            out_store(j - 2, p).wait()

        rows_v, self_v, out_v = rows[p], selfs[p], outs[p]
        for c in range(CHUNK):
            for dk in range(D // 16):
                sl = pl.ds(dk * 16, 16)
                acc = self_v[c, sl]
                for s in range(S):
                    acc = acc + rows_v[c * S + s, sl]
                out_v[c, sl] = acc

        out_store(j, p).start()

        @pl.when(j + 2 < nchunks)
        def _():
            fire(j + 2, p)

    fire(0, 0)
    fire(1, 1)

    nsteps = (nchunks + 1) // 2

    def body2(jj, _):
        j0 = jj * 2
        consume(j0, jj, 0)

        @pl.when(j0 + 1 < nchunks)
        def _():
            consume(j0 + 1, jj, 1)

        return 0

    lax.fori_loop(0, nsteps, body2, 0)

    out_store(2 * ((nchunks + 1) // 2 - 1), 0).wait()
    out_store(2 * (nchunks // 2) - 1, 1).wait()


_gather_sum = functools.partial(
    pl.kernel,
    out_type=jax.ShapeDtypeStruct((N, D), jnp.float32),
    mesh=plsc.VectorSubcoreMesh(core_axis_name="c", subcore_axis_name="s"),
    scratch_types=[
        pltpu.VMEM((MAXC, CHUNK * S), jnp.int32),
        pltpu.VMEM((CHUNK * S, D), jnp.float32),
        pltpu.VMEM((CHUNK * S, D), jnp.float32),
        pltpu.VMEM((CHUNK, D), jnp.float32),
        pltpu.VMEM((CHUNK, D), jnp.float32),
        pltpu.VMEM((CHUNK, D), jnp.float32),
        pltpu.VMEM((CHUNK, D), jnp.float32),
        pltpu.SemaphoreType.DMA,
        pltpu.SemaphoreType.DMA,
        pltpu.SemaphoreType.DMA,
        pltpu.SemaphoreType.DMA,
        pltpu.SemaphoreType.DMA,
        pltpu.SemaphoreType.DMA,
    ],
)(_gather_sum_body)


BN = 1000


def _norm_linear_body(h_ref, wt_ref, b_ref, out_ref):
    h = h_ref[...]
    ss = jnp.sum(h * h, axis=1, keepdims=True)
    denom = jnp.maximum(jnp.sqrt(ss), 1e-12)
    hn = h / denom
    out_ref[...] = (
        jnp.dot(hn, wt_ref[...], preferred_element_type=jnp.float32)
        + b_ref[...]
    )


def _norm_linear(h, wt, b2d):
    return pl.pallas_call(
        _norm_linear_body,
        grid=(N // BN,),
        in_specs=[
            pl.BlockSpec((BN, D), lambda i: (i, 0)),
            pl.BlockSpec((D, D), lambda i: (0, 0)),
            pl.BlockSpec((1, D), lambda i: (0, 0)),
        ],
        out_specs=pl.BlockSpec((BN, D), lambda i: (i, 0)),
        out_shape=jax.ShapeDtypeStruct((N, D), jnp.float32),
    )(h, wt, b2d)


def kernel(features, non_neighbor_idx, W, b):
    idx = non_neighbor_idx.astype(jnp.int32).reshape(TOTAL_CHUNKS, CHUNK * S)
    idx = jnp.pad(idx, ((0, PADC - TOTAL_CHUNKS), (0, 0)))
    h = _gather_sum(features, idx)
    return _norm_linear(h, W.T, b.reshape(1, D))


# speedup vs baseline: 4.9779x; 1.4126x over previous
import functools

import jax
import jax.numpy as jnp
from jax import lax
from jax.experimental import pallas as pl
from jax.experimental.pallas import tpu as pltpu
from jax.experimental.pallas import tpu_sc as plsc

N = 100000
D = 128
S = 16

NC = 2
NS = 16
NW = NC * NS

CHUNK = 8
TOTAL_CHUNKS = N // CHUNK
Q, R = divmod(TOTAL_CHUNKS, NW)
MAXC = 400
PADC = 12504
NBUF = 4


def _gather_sum_body(feat_hbm, idx_hbm, out_hbm, idx_v,
                     rows0, rows1, rows2, rows3, out0, out1, out2, out3,
                     sr0, sr1, sr2, sr3, so0, so1, so2, so3):
    wid = lax.axis_index("s") * NC + lax.axis_index("c")
    chunk_base = wid * Q + lax.min(wid, R)
    nchunks = Q + jnp.where(wid < R, 1, 0)

    pbase = (chunk_base // 8) * 8
    off = chunk_base - pbase
    pltpu.sync_copy(idx_hbm.at[pl.ds(pbase, MAXC)], idx_v)

    rows = (rows0, rows1, rows2, rows3)
    outs = (out0, out1, out2, out3)
    sems_r = (sr0, sr1, sr2, sr3)
    sems_o = (so0, so1, so2, so3)

    def gather(j, p):
        return pltpu.make_async_copy(
            feat_hbm.at[idx_v.at[off + j]], rows[p], sems_r[p])

    def out_store(j, p):
        node_base = (chunk_base + j) * CHUNK
        return pltpu.make_async_copy(
            outs[p], out_hbm.at[pl.ds(node_base, CHUNK)], sems_o[p])

    def consume(j, jj, p):
        gather(j, p).wait()

        @pl.when(jj >= 1)
        def _():
            out_store(j - NBUF, p).wait()

        rows_v, out_v = rows[p], outs[p]

        def node_body(c, _):
            for dk in range(D // 16):
                sl = pl.ds(dk * 16, 16)
                acc = rows_v[c * S, sl]
                for s in range(1, S):
                    acc = acc + rows_v[c * S + s, sl]
                out_v[c, sl] = acc
            return 0

        lax.fori_loop(0, CHUNK, node_body, 0, unroll=2)

        out_store(j, p).start()

        @pl.when(j + NBUF < nchunks)
        def _():
            gather(j + NBUF, p).start()

    for p in range(NBUF):
        gather(p, p).start()

    nsteps = (nchunks + NBUF - 1) // NBUF

    def body4(jj, _):
        j0 = jj * NBUF
        consume(j0, jj, 0)
        for p in range(1, NBUF):

            @pl.when(j0 + p < nchunks)
            def _(p=p):
                consume(j0 + p, jj, p)

        return 0

    lax.fori_loop(0, nsteps, body4, 0)

    for p in range(NBUF):
        out_store(((nchunks - 1 - p) // NBUF) * NBUF + p, p).wait()


_gather_sum = functools.partial(
    pl.kernel,
    out_type=jax.ShapeDtypeStruct((N, D), jnp.float32),
    mesh=plsc.VectorSubcoreMesh(core_axis_name="c", subcore_axis_name="s"),
    scratch_types=(
        [pltpu.VMEM((MAXC, CHUNK * S), jnp.int32)]
        + [pltpu.VMEM((CHUNK * S, D), jnp.float32) for _ in range(NBUF)]
        + [pltpu.VMEM((CHUNK, D), jnp.float32) for _ in range(NBUF)]
        + [pltpu.SemaphoreType.DMA for _ in range(2 * NBUF)]
    ),
)(_gather_sum_body)


BN = 1000


def _norm_linear_body(h_ref, f_ref, wt_ref, b_ref, out_ref):
    h = h_ref[...] + f_ref[...]
    ss = jnp.sum(h * h, axis=1, keepdims=True)
    denom = jnp.maximum(jnp.sqrt(ss), 1e-12)
    hn = h / denom
    out_ref[...] = (
        jnp.dot(hn, wt_ref[...], preferred_element_type=jnp.float32)
        + b_ref[...]
    )


def _norm_linear(h, features, wt, b2d):
    return pl.pallas_call(
        _norm_linear_body,
        grid=(N // BN,),
        in_specs=[
            pl.BlockSpec((BN, D), lambda i: (i, 0)),
            pl.BlockSpec((BN, D), lambda i: (i, 0)),
            pl.BlockSpec((D, D), lambda i: (0, 0)),
            pl.BlockSpec((1, D), lambda i: (0, 0)),
        ],
        out_specs=pl.BlockSpec((BN, D), lambda i: (i, 0)),
        out_shape=jax.ShapeDtypeStruct((N, D), jnp.float32),
    )(h, features, wt, b2d)


def kernel(features, non_neighbor_idx, W, b):
    idx = non_neighbor_idx.astype(jnp.int32).reshape(TOTAL_CHUNKS, CHUNK * S)
    idx = jnp.pad(idx, ((0, PADC - TOTAL_CHUNKS), (0, 0)))
    h = _gather_sum(features, idx)
    return _norm_linear(h, features, W.T, b.reshape(1, D))


# speedup vs baseline: 6.0745x; 1.2203x over previous
import functools

import jax
import jax.numpy as jnp
from jax import lax
from jax.experimental import pallas as pl
from jax.experimental.pallas import tpu as pltpu
from jax.experimental.pallas import tpu_sc as plsc

N = 100000
D = 128
S = 16

NC = 2
NS = 16
NW = NC * NS

CHUNK = 8
TOTAL_CHUNKS = N // CHUNK
Q, R = divmod(TOTAL_CHUNKS, NW)
MAXC = 400
PADC = 12504
NBUF = 4


def _gather_sum_body(feat_hbm, idx_hbm, out_hbm, idx_v,
                     rows0, rows1, rows2, rows3, out0, out1, out2, out3,
                     sr0, sr1, sr2, sr3, so0, so1, so2, so3):
    wid = lax.axis_index("s") * NC + lax.axis_index("c")
    chunk_base = wid * Q + lax.min(wid, R)
    nchunks = Q + jnp.where(wid < R, 1, 0)

    pbase = (chunk_base // 8) * 8
    off = chunk_base - pbase
    pltpu.sync_copy(idx_hbm.at[pl.ds(pbase, MAXC)], idx_v)

    rows = (rows0, rows1, rows2, rows3)
    outs = (out0, out1, out2, out3)
    sems_r = (sr0, sr1, sr2, sr3)
    sems_o = (so0, so1, so2, so3)

    def gather(j, p):
        return pltpu.make_async_copy(
            feat_hbm.at[idx_v.at[off + j]], rows[p], sems_r[p])

    def out_store(j, p):
        node_base = (chunk_base + j) * CHUNK
        return pltpu.make_async_copy(
            outs[p], out_hbm.at[pl.ds(node_base, CHUNK)], sems_o[p])

    def consume(j, jj, p):
        gather(j, p).wait()

        @pl.when(jj >= 1)
        def _():
            out_store(j - NBUF, p).wait()

        rows_v, out_v = rows[p], outs[p]

        def node_body(c, _):
            for dk in range(D // 16):
                sl = pl.ds(dk * 16, 16)
                v = [rows_v[c * S + s, sl] for s in range(S)]
                while len(v) > 1:
                    v = [v[i] + v[i + 1] for i in range(0, len(v), 2)]
                out_v[c, sl] = v[0]
            return 0

        lax.fori_loop(0, CHUNK, node_body, 0, unroll=2)

        out_store(j, p).start()

        @pl.when(j + NBUF < nchunks)
        def _():
            gather(j + NBUF, p).start()

    for p in range(NBUF):
        gather(p, p).start()

    nsteps = (nchunks + NBUF - 1) // NBUF

    def body4(jj, _):
        j0 = jj * NBUF
        consume(j0, jj, 0)
        for p in range(1, NBUF):

            @pl.when(j0 + p < nchunks)
            def _(p=p):
                consume(j0 + p, jj, p)

        return 0

    lax.fori_loop(0, nsteps, body4, 0)

    for p in range(NBUF):
        out_store(((nchunks - 1 - p) // NBUF) * NBUF + p, p).wait()


_gather_sum = functools.partial(
    pl.kernel,
    out_type=jax.ShapeDtypeStruct((N, D), jnp.float32),
    mesh=plsc.VectorSubcoreMesh(core_axis_name="c", subcore_axis_name="s"),
    scratch_types=(
        [pltpu.VMEM((MAXC, CHUNK * S), jnp.int32)]
        + [pltpu.VMEM((CHUNK * S, D), jnp.float32) for _ in range(NBUF)]
        + [pltpu.VMEM((CHUNK, D), jnp.float32) for _ in range(NBUF)]
        + [pltpu.SemaphoreType.DMA for _ in range(2 * NBUF)]
    ),
)(_gather_sum_body)


BN = 1000


def _norm_linear_body(h_ref, f_ref, wt_ref, b_ref, out_ref):
    h = h_ref[...] + f_ref[...]
    ss = jnp.sum(h * h, axis=1, keepdims=True)
    denom = jnp.maximum(jnp.sqrt(ss), 1e-12)
    hn = h / denom
    out_ref[...] = (
        jnp.dot(hn, wt_ref[...], preferred_element_type=jnp.float32)
        + b_ref[...]
    )


def _norm_linear(h, features, wt, b2d):
    return pl.pallas_call(
        _norm_linear_body,
        grid=(N // BN,),
        in_specs=[
            pl.BlockSpec((BN, D), lambda i: (i, 0)),
            pl.BlockSpec((BN, D), lambda i: (i, 0)),
            pl.BlockSpec((D, D), lambda i: (0, 0)),
            pl.BlockSpec((1, D), lambda i: (0, 0)),
        ],
        out_specs=pl.BlockSpec((BN, D), lambda i: (i, 0)),
        out_shape=jax.ShapeDtypeStruct((N, D), jnp.float32),
    )(h, features, wt, b2d)


def kernel(features, non_neighbor_idx, W, b):
    idx = non_neighbor_idx.astype(jnp.int32).reshape(TOTAL_CHUNKS, CHUNK * S)
    idx = jnp.pad(idx, ((0, PADC - TOTAL_CHUNKS), (0, 0)))
    h = _gather_sum(features, idx)
    return _norm_linear(h, features, W.T, b.reshape(1, D))


# speedup vs baseline: 7.9233x; 1.3043x over previous
import functools

import jax
import jax.numpy as jnp
from jax import lax
from jax.experimental import pallas as pl
from jax.experimental.pallas import tpu as pltpu
from jax.experimental.pallas import tpu_sc as plsc

N = 100000
D = 128
S = 16

NC = 2
NS = 16
NW = NC * NS

SUPER = 128
TOTAL_SUPER = (N + SUPER - 1) // SUPER
TAIL = N - (TOTAL_SUPER - 1) * SUPER
NPAD = TOTAL_SUPER * SUPER
QS, RS = divmod(TOTAL_SUPER, NW)
MAXC = 400
IDX_ROWS = TOTAL_SUPER * S
NBUF = 4


def _gather_sum_body(feat_hbm, idx_hbm, out_hbm, idx_v,
                     rows0, rows1, rows2, rows3,
                     sr0, sr1, sr2, sr3, so0, so1, so2, so3):
    wid = lax.axis_index("s") * NC + lax.axis_index("c")
    super_start = wid * QS + lax.min(wid, RS)
    nsupers = QS + jnp.where(wid < RS, 1, 0)

    pbase = lax.min(super_start * S, IDX_ROWS - MAXC)
    off = super_start * S - pbase
    pltpu.sync_copy(idx_hbm.at[pl.ds(pbase, MAXC)], idx_v)

    rows = (rows0, rows1, rows2, rows3)
    sems_r = (sr0, sr1, sr2, sr3)
    sems_o = (so0, so1, so2, so3)

    def zero_buf(p):
        z = jnp.zeros((16,), jnp.float32)

        def zb(r, _):
            for dk in range(D // 16):
                rows[p][r, pl.ds(dk * 16, 16)] = z
            return 0

        lax.fori_loop(0, SUPER, zb, 0, unroll=2)

    def fire_streams(j, p):
        for s in range(S):
            pltpu.async_copy(
                feat_hbm.at[idx_v.at[off + j * S + s]], rows[p], sems_r[p],
                add=True)

    def drain_streams(j, p):
        for s in range(S):
            pltpu.make_async_copy(
                feat_hbm.at[idx_v.at[off + j * S + s]], rows[p],
                sems_r[p]).wait()

    def store_full(j, p):
        node_base = (super_start + j) * SUPER
        return pltpu.make_async_copy(
            rows[p], out_hbm.at[pl.ds(node_base, SUPER)], sems_o[p])

    def store_tail(j, p):
        node_base = (super_start + j) * SUPER
        return pltpu.make_async_copy(
            rows[p].at[pl.ds(0, TAIL)],
            out_hbm.at[pl.ds(node_base, TAIL)], sems_o[p])

    def start_store(j, p):
        is_tail = super_start + j == TOTAL_SUPER - 1

        @pl.when(jnp.logical_not(is_tail))
        def _():
            store_full(j, p).start()

        @pl.when(is_tail)
        def _():
            store_tail(j, p).start()

    def consume(j, p, pm1):
        drain_streams(j, p)
        start_store(j, p)

        @pl.when((j >= 1) & (j - 1 + NBUF < nsupers))
        def _():
            store_full(j - 1, pm1).wait()
            zero_buf(pm1)
            fire_streams(j - 1 + NBUF, pm1)

    for p in range(NBUF):
        zero_buf(p)
        fire_streams(p, p)

    nsteps = (nsupers + NBUF - 1) // NBUF

    def body(jj, _):
        j0 = jj * NBUF
        consume(j0, 0, NBUF - 1)
        for p in range(1, NBUF):

            @pl.when(j0 + p < nsupers)
            def _(p=p):
                consume(j0 + p, p, p - 1)

        return 0

    lax.fori_loop(0, nsteps, body, 0)

    for p in range(NBUF):
        j = ((nsupers - 1 - p) // NBUF) * NBUF + p
        is_tail = super_start + j == TOTAL_SUPER - 1

        @pl.when(jnp.logical_not(is_tail))
        def _(j=j, p=p):
            store_full(j, p).wait()

        @pl.when(is_tail)
        def _(j=j, p=p):
            store_tail(j, p).wait()


_gather_sum = functools.partial(
    pl.kernel,
    out_type=jax.ShapeDtypeStruct((N, D), jnp.float32),
    mesh=plsc.VectorSubcoreMesh(core_axis_name="c", subcore_axis_name="s"),
    scratch_types=(
        [pltpu.VMEM((MAXC, SUPER), jnp.int32)]
        + [pltpu.VMEM((SUPER, D), jnp.float32) for _ in range(NBUF)]
        + [pltpu.SemaphoreType.DMA for _ in range(2 * NBUF)]
    ),
)(_gather_sum_body)


BN = 1000


def _norm_linear_body(h_ref, f_ref, wt_ref, b_ref, out_ref):
    h = h_ref[...] + f_ref[...]
    ss = jnp.sum(h * h, axis=1, keepdims=True)
    denom = jnp.maximum(jnp.sqrt(ss), 1e-12)
    hn = h / denom
    out_ref[...] = (
        jnp.dot(hn, wt_ref[...], preferred_element_type=jnp.float32)
        + b_ref[...]
    )


def _norm_linear(h, features, wt, b2d):
    return pl.pallas_call(
        _norm_linear_body,
        grid=(N // BN,),
        in_specs=[
            pl.BlockSpec((BN, D), lambda i: (i, 0)),
            pl.BlockSpec((BN, D), lambda i: (i, 0)),
            pl.BlockSpec((D, D), lambda i: (0, 0)),
            pl.BlockSpec((1, D), lambda i: (0, 0)),
        ],
        out_specs=pl.BlockSpec((BN, D), lambda i: (i, 0)),
        out_shape=jax.ShapeDtypeStruct((N, D), jnp.float32),
    )(h, features, wt, b2d)


def kernel(features, non_neighbor_idx, W, b):
    idx = non_neighbor_idx.astype(jnp.int32)
    idx = jnp.pad(idx, ((0, NPAD - N), (0, 0)))
    idx_t = idx.reshape(TOTAL_SUPER, SUPER, S).transpose(0, 2, 1)
    idx_t = idx_t.reshape(IDX_ROWS, SUPER)
    h = _gather_sum(features, idx_t)
    return _norm_linear(h, features, W.T, b.reshape(1, D))


# speedup vs baseline: 8.4068x; 1.0610x over previous
import functools

import jax
import jax.numpy as jnp
from jax import lax
from jax.experimental import pallas as pl
from jax.experimental.pallas import tpu as pltpu
from jax.experimental.pallas import tpu_sc as plsc

N = 100000
D = 128
S = 16

NC = 2
NS = 16
NW = NC * NS

SUPER = 128
TOTAL_SUPER = (N + SUPER - 1) // SUPER
TAIL = N - (TOTAL_SUPER - 1) * SUPER
NPAD = TOTAL_SUPER * SUPER
T0 = 352
Q0, R0 = divmod(T0, NS)
Q1, R1 = divmod(TOTAL_SUPER - T0, NS)
MAXSUP = max(Q0, Q1) + 1
MAXC = ((MAXSUP * S + 7) // 8) * 8
IDX_ROWS = TOTAL_SUPER * S
NBUF = 4


def _gather_sum_body(feat_hbm, idx_hbm, out_hbm, idx_v,
                     rows0, rows1, rows2, rows3,
                     sr0, sr1, sr2, sr3, so0, so1, so2, so3):
    c = lax.axis_index("c")
    s = lax.axis_index("s")
    super_start = jnp.where(
        c == 0,
        s * Q0 + lax.min(s, R0),
        T0 + s * Q1 + lax.min(s, R1),
    )
    nsupers = jnp.where(c == 0,
                        Q0 + jnp.where(s < R0, 1, 0),
                        Q1 + jnp.where(s < R1, 1, 0))

    pbase = lax.min(super_start * S, IDX_ROWS - MAXC)
    off = super_start * S - pbase
    pltpu.sync_copy(idx_hbm.at[pl.ds(pbase, MAXC)], idx_v)

    rows = (rows0, rows1, rows2, rows3)
    sems_r = (sr0, sr1, sr2, sr3)
    sems_o = (so0, so1, so2, so3)

    def zero_buf(p):
        z = jnp.zeros((16,), jnp.float32)

        def zb(r, _):
            for dk in range(D // 16):
                rows[p][r, pl.ds(dk * 16, 16)] = z
            return 0

        lax.fori_loop(0, SUPER, zb, 0, unroll=2)

    def fire_streams(j, p):
        for s in range(S):
            pltpu.async_copy(
                feat_hbm.at[idx_v.at[off + j * S + s]], rows[p], sems_r[p],
                add=True)

    def drain_streams(j, p):
        for s in range(S):
            pltpu.make_async_copy(
                feat_hbm.at[idx_v.at[off + j * S + s]], rows[p],
                sems_r[p]).wait()

    def store_full(j, p):
        node_base = (super_start + j) * SUPER
        return pltpu.make_async_copy(
            rows[p], out_hbm.at[pl.ds(node_base, SUPER)], sems_o[p])

    def store_tail(j, p):
        node_base = (super_start + j) * SUPER
        return pltpu.make_async_copy(
            rows[p].at[pl.ds(0, TAIL)],
            out_hbm.at[pl.ds(node_base, TAIL)], sems_o[p])

    def start_store(j, p):
        is_tail = super_start + j == TOTAL_SUPER - 1

        @pl.when(jnp.logical_not(is_tail))
        def _():
            store_full(j, p).start()

        @pl.when(is_tail)
        def _():
            store_tail(j, p).start()

    def consume(j, p, pm1):
        drain_streams(j, p)
        start_store(j, p)

        @pl.when((j >= 1) & (j - 1 + NBUF < nsupers))
        def _():
            store_full(j - 1, pm1).wait()
            zero_buf(pm1)
            fire_streams(j - 1 + NBUF, pm1)

    for p in range(NBUF):
        zero_buf(p)
        fire_streams(p, p)

    nsteps = (nsupers + NBUF - 1) // NBUF

    def body(jj, _):
        j0 = jj * NBUF
        consume(j0, 0, NBUF - 1)
        for p in range(1, NBUF):

            @pl.when(j0 + p < nsupers)
            def _(p=p):
                consume(j0 + p, p, p - 1)

        return 0

    lax.fori_loop(0, nsteps, body, 0)

    for p in range(NBUF):
        j = ((nsupers - 1 - p) // NBUF) * NBUF + p
        is_tail = super_start + j == TOTAL_SUPER - 1

        @pl.when(jnp.logical_not(is_tail))
        def _(j=j, p=p):
            store_full(j, p).wait()

        @pl.when(is_tail)
        def _(j=j, p=p):
            store_tail(j, p).wait()


_gather_sum = functools.partial(
    pl.kernel,
    out_type=jax.ShapeDtypeStruct((N, D), jnp.float32),
    mesh=plsc.VectorSubcoreMesh(core_axis_name="c", subcore_axis_name="s"),
    scratch_types=(
        [pltpu.VMEM((MAXC, SUPER), jnp.int32)]
        + [pltpu.VMEM((SUPER, D), jnp.float32) for _ in range(NBUF)]
        + [pltpu.SemaphoreType.DMA for _ in range(2 * NBUF)]
    ),
)(_gather_sum_body)


BN = 5000


def _norm_linear_body(h_ref, f_ref, wt_ref, b_ref, out_ref):
    h = h_ref[...] + f_ref[...]
    ss = jnp.sum(h * h, axis=1, keepdims=True)
    denom = jnp.maximum(jnp.sqrt(ss), 1e-12)
    hn = h / denom
    out_ref[...] = (
        jnp.dot(hn, wt_ref[...], preferred_element_type=jnp.float32)
        + b_ref[...]
    )


def _norm_linear(h, features, wt, b2d):
    return pl.pallas_call(
        _norm_linear_body,
        grid=(N // BN,),
        in_specs=[
            pl.BlockSpec((BN, D), lambda i: (i, 0)),
            pl.BlockSpec((BN, D), lambda i: (i, 0)),
            pl.BlockSpec((D, D), lambda i: (0, 0)),
            pl.BlockSpec((1, D), lambda i: (0, 0)),
        ],
        out_specs=pl.BlockSpec((BN, D), lambda i: (i, 0)),
        out_shape=jax.ShapeDtypeStruct((N, D), jnp.float32),
    )(h, features, wt, b2d)


def kernel(features, non_neighbor_idx, W, b):
    idx = non_neighbor_idx.astype(jnp.int32)
    idx = jnp.pad(idx, ((0, NPAD - N), (0, 0)))
    idx_t = idx.reshape(TOTAL_SUPER, SUPER, S).transpose(0, 2, 1)
    idx_t = idx_t.reshape(IDX_ROWS, SUPER)
    h = _gather_sum(features, idx_t)
    return _norm_linear(h, features, W.T, b.reshape(1, D))


# speedup vs baseline: 8.8965x; 1.0582x over previous
import functools

import jax
import jax.numpy as jnp
from jax import lax
from jax.experimental import pallas as pl
from jax.experimental.pallas import tpu as pltpu
from jax.experimental.pallas import tpu_sc as plsc

N = 100000
D = 128
S = 16

NC = 2
NS = 16
NW = NC * NS

SUPER = 128
TOTAL_SUPER = (N + SUPER - 1) // SUPER
TAIL = N - (TOTAL_SUPER - 1) * SUPER
NPAD = TOTAL_SUPER * SUPER
T0 = 422
Q0, R0 = divmod(T0, NS)
Q1, R1 = divmod(TOTAL_SUPER - T0, NS)
MAXSUP = max(Q0, Q1) + 1
MAXC = ((MAXSUP * S + 7) // 8) * 8
IDX_ROWS = TOTAL_SUPER * S
NBUF = 4


def _gather_sum_body(feat_hbm, idx_hbm, out_hbm, idx_v,
                     rows0, rows1, rows2, rows3,
                     sr0, sr1, sr2, sr3, so0, so1, so2, so3):
    c = lax.axis_index("c")
    s = lax.axis_index("s")
    super_start = jnp.where(
        c == 0,
        s * Q0 + lax.min(s, R0),
        T0 + s * Q1 + lax.min(s, R1),
    )
    nsupers = jnp.where(c == 0,
                        Q0 + jnp.where(s < R0, 1, 0),
                        Q1 + jnp.where(s < R1, 1, 0))

    pbase = lax.min(super_start * S, IDX_ROWS - MAXC)
    off = super_start * S - pbase
    pltpu.sync_copy(idx_hbm.at[pl.ds(pbase, MAXC)], idx_v)

    rows = (rows0, rows1, rows2, rows3)
    sems_r = (sr0, sr1, sr2, sr3)
    sems_o = (so0, so1, so2, so3)

    def zero_buf(p):
        z = jnp.zeros((16,), jnp.float32)

        def zb(r, _):
            for dk in range(D // 16):
                rows[p][r, pl.ds(dk * 16, 16)] = z
            return 0

        lax.fori_loop(0, SUPER, zb, 0, unroll=2)

    def fire_streams(j, p):
        for s in range(S):
            pltpu.async_copy(
                feat_hbm.at[idx_v.at[off + j * S + s]], rows[p], sems_r[p],
                add=True)

    def drain_streams(j, p):
        for s in range(S):
            pltpu.make_async_copy(
                feat_hbm.at[idx_v.at[off + j * S + s]], rows[p],
                sems_r[p]).wait()

    def store_full(j, p):
        node_base = (super_start + j) * SUPER
        return pltpu.make_async_copy(
            rows[p], out_hbm.at[pl.ds(node_base, SUPER)], sems_o[p])

    def store_tail(j, p):
        node_base = (super_start + j) * SUPER
        return pltpu.make_async_copy(
            rows[p].at[pl.ds(0, TAIL)],
            out_hbm.at[pl.ds(node_base, TAIL)], sems_o[p])

    def start_store(j, p):
        is_tail = super_start + j == TOTAL_SUPER - 1

        @pl.when(jnp.logical_not(is_tail))
        def _():
            store_full(j, p).start()

        @pl.when(is_tail)
        def _():
            store_tail(j, p).start()

    def consume(j, p, pm1):
        drain_streams(j, p)
        start_store(j, p)

        @pl.when((j >= 1) & (j - 1 + NBUF < nsupers))
        def _():
            store_full(j - 1, pm1).wait()
            zero_buf(pm1)
            fire_streams(j - 1 + NBUF, pm1)

    for p in range(NBUF):
        zero_buf(p)
        fire_streams(p, p)

    nsteps = (nsupers + NBUF - 1) // NBUF

    def body(jj, _):
        j0 = jj * NBUF
        consume(j0, 0, NBUF - 1)
        for p in range(1, NBUF):

            @pl.when(j0 + p < nsupers)
            def _(p=p):
                consume(j0 + p, p, p - 1)

        return 0

    lax.fori_loop(0, nsteps, body, 0)

    for p in range(NBUF):
        j = ((nsupers - 1 - p) // NBUF) * NBUF + p
        is_tail = super_start + j == TOTAL_SUPER - 1

        @pl.when(jnp.logical_not(is_tail))
        def _(j=j, p=p):
            store_full(j, p).wait()

        @pl.when(is_tail)
        def _(j=j, p=p):
            store_tail(j, p).wait()


_gather_sum = functools.partial(
    pl.kernel,
    out_type=jax.ShapeDtypeStruct((N, D), jnp.float32),
    mesh=plsc.VectorSubcoreMesh(core_axis_name="c", subcore_axis_name="s"),
    scratch_types=(
        [pltpu.VMEM((MAXC, SUPER), jnp.int32)]
        + [pltpu.VMEM((SUPER, D), jnp.float32) for _ in range(NBUF)]
        + [pltpu.SemaphoreType.DMA for _ in range(2 * NBUF)]
    ),
)(_gather_sum_body)


BN = 5000


def _norm_linear_body(h_ref, f_ref, wt_ref, b_ref, out_ref):
    h = h_ref[...] + f_ref[...]
    ss = jnp.sum(h * h, axis=1, keepdims=True)
    denom = jnp.maximum(jnp.sqrt(ss), 1e-12)
    hn = h / denom
    out_ref[...] = (
        jnp.dot(hn, wt_ref[...], preferred_element_type=jnp.float32)
        + b_ref[...]
    )


def _norm_linear(h, features, wt, b2d):
    return pl.pallas_call(
        _norm_linear_body,
        grid=(N // BN,),
        in_specs=[
            pl.BlockSpec((BN, D), lambda i: (i, 0)),
            pl.BlockSpec((BN, D), lambda i: (i, 0)),
            pl.BlockSpec((D, D), lambda i: (0, 0)),
            pl.BlockSpec((1, D), lambda i: (0, 0)),
        ],
        out_specs=pl.BlockSpec((BN, D), lambda i: (i, 0)),
        out_shape=jax.ShapeDtypeStruct((N, D), jnp.float32),
    )(h, features, wt, b2d)


def kernel(features, non_neighbor_idx, W, b):
    idx = non_neighbor_idx.astype(jnp.int32)
    idx = jnp.pad(idx, ((0, NPAD - N), (0, 0)))
    idx_t = idx.reshape(TOTAL_SUPER, SUPER, S).transpose(0, 2, 1)
    idx_t = idx_t.reshape(IDX_ROWS, SUPER)
    h = _gather_sum(features, idx_t)
    return _norm_linear(h, features, W.T, b.reshape(1, D))
